# baseline (speedup 1.0000x reference)
import functools

import jax
import jax.numpy as jnp
from jax import lax
from jax.experimental import pallas as pl
from jax.experimental.pallas import tpu as pltpu
from jax.experimental.pallas import tpu_sc as plsc

_VOCAB = 100000
_EMB = 128
_BATCH = 4096
_CTX = 50
_NNEG = 20
_CTXP = 56
_ROWS = 784
_VPAD = _ROWS * 128
_CPAD = 1024
_LASTB = 781
_NW = 32
_BPW = _BATCH // _NW
_DPW = _BPW * _NNEG


def _prep_body(w_ref, is_ref, cdf_ref, coarse_ref, u_ref, inp_ref):
    w = w_ref[:]
    f32 = jnp.float32
    r = lax.broadcasted_iota(jnp.int32, (128, 128), 0)
    c = lax.broadcasted_iota(jnp.int32, (128, 128), 1)
    tri = (r <= c).astype(f32)
    rowpref = jnp.dot(w, tri, preferred_element_type=f32)
    rowtot = rowpref[:, 127:128]
    rr = lax.broadcasted_iota(jnp.int32, (_ROWS, _ROWS), 0)
    cc = lax.broadcasted_iota(jnp.int32, (_ROWS, _ROWS), 1)
    lo = (cc <= rr).astype(f32)
    rowcum = jnp.dot(lo, rowtot, preferred_element_type=f32)
    cdf_ref[:] = rowpref + (rowcum - rowtot)
    coarse_ref[:] = jnp.concatenate(
        [rowcum, jnp.full((_CPAD - _ROWS, 1), 2.0, f32)], axis=0)
    pltpu.prng_seed(42)
    bits = pltpu.prng_random_bits((_BATCH, 32))
    bits = lax.bitcast_convert_type(bits, jnp.int32) & jnp.int32(0x7FFFFFFF)
    u_ref[:] = (bits.astype(f32) * f32(2.0 ** -31))[:, :_NNEG]
    isv = is_ref[:]
    inp_ref[:] = jnp.concatenate([isv, isv[:, : _CTXP - _CTX]], axis=1)


_prep = pl.pallas_call(
    _prep_body,
    out_shape=[
        jax.ShapeDtypeStruct((_ROWS, 128), jnp.float32),
        jax.ShapeDtypeStruct((_CPAD, 1), jnp.float32),
        jax.ShapeDtypeStruct((_BATCH, _NNEG), jnp.float32),
        jax.ShapeDtypeStruct((_BATCH, _CTXP), jnp.int32),
    ],
)


def _sc_body(table, inp, uflat, mw, coarse, fine2d, odot_hbm, ndot_hbm,
             inp_v, u_v, mw_v, coarse_v, buck_v, seg_v, nidx_v,
             mwrows_v, ctx_v, neg_v, odot_v, ndst_v,
             mw_sem, fsem0, fsem1, csem0, csem1, nsem0, nsem1, wsem0, wsem1):
    f32 = jnp.float32
    i32 = jnp.int32
    wid = lax.axis_index("s") * 2 + lax.axis_index("c")
    fsem = (fsem0, fsem1)
    csem = (csem0, csem1)
    nsem = (nsem0, nsem1)
    wsem = (wsem0, wsem1)

    pltpu.sync_copy(inp.at[pl.ds(wid * (_BPW * _CTXP), _BPW * _CTXP)], inp_v)
    pltpu.sync_copy(uflat.at[pl.ds(wid * _DPW, _DPW)], u_v)
    pltpu.sync_copy(mw.at[pl.ds(wid * _BPW, _BPW)], mw_v)
    pltpu.sync_copy(coarse, coarse_v)

    pltpu.async_copy(table.at[mw_v], mwrows_v, mw_sem)

    def coarse_chunk(v, _):
        base = pl.multiple_of(v * 16, 16)
        u16 = u_v[pl.ds(base, 16)]
        pos = jnp.zeros((16,), i32)
        s = _CPAD // 2
        while s >= 1:
            probe = pos + (s - 1)
            val = plsc.load_gather(coarse_v, [probe])
            pos = pos + jnp.where(val <= u16, i32(s), i32(0))
            s //= 2
        buck_v[pl.ds(base, 16)] = jnp.minimum(pos, i32(_LASTB))
        return 0

    lax.fori_loop(0, _DPW // 16, coarse_chunk, 0)

    nchunk = _DPW // 128

    def start_fine(cI, b):
        base = pl.multiple_of(cI * 128, 128)
        pltpu.async_copy(fine2d.at[buck_v.at[pl.ds(base, 128)]],
                         seg_v.at[b], fsem[b])

    start_fine(0, 0)

    def fine_outer(f2, _):
        for b in range(2):
            cI = f2 * 2 + b
            start_fine(jnp.minimum(cI + 1, nchunk - 1), 1 - b)
            pltpu.make_async_copy(fine2d.at[buck_v.at[pl.ds(0, 128)]],
                                  seg_v.at[b], fsem[b]).wait()
            base = pl.multiple_of(cI * 128, 128)
            for v in range(8):
                off = pl.multiple_of(base + v * 16, 16)
                u16 = u_v[pl.ds(off, 16)]
                b16 = buck_v[pl.ds(off, 16)]
                rows = v * 16 + lax.iota(i32, 16)
                k = jnp.zeros((16,), i32)
                s = 64
                while s >= 1:
                    probe = k + (s - 1)
                    val = plsc.load_gather(seg_v.at[b], [rows, probe])
                    k = k + jnp.where(val <= u16, i32(s), i32(0))
                    s //= 2
                nidx_v[pl.ds(off, 16)] = jnp.minimum(b16 * 128 + k,
                                                     i32(_VOCAB - 1))
        return 0

    lax.fori_loop(0, nchunk // 2, fine_outer, 0)
    pltpu.make_async_copy(fine2d.at[buck_v.at[pl.ds(0, 128)]],
                          seg_v.at[0], fsem[0]).wait()

    lane = lax.iota(i32, 16)
    npair = _BPW // 2

    def start_pair(p, b):
        nb = pl.multiple_of(p * (2 * _NNEG), 8)
        for e in range(2):
            cb = pl.multiple_of(p * (2 * _CTXP) + e * _CTXP, 8)
            pltpu.async_copy(table.at[inp_v.at[pl.ds(cb, _CTX)]],
                             ctx_v.at[b, pl.ds(e * _CTXP, _CTX)], csem[b])
        pltpu.async_copy(table.at[nidx_v.at[pl.ds(nb, 2 * _NNEG)]],
                         neg_v.at[b], nsem[b])

    start_pair(0, 0)
    pltpu.make_async_copy(table.at[mw_v], mwrows_v, mw_sem).wait()

    def pair_outer(p2, _):
        for b in range(2):
            p = p2 * 2 + b
            start_pair(jnp.minimum(p + 1, npair - 1), 1 - b)
            for _e in range(2):
                pltpu.make_async_copy(
                    table.at[inp_v.at[pl.ds(0, _CTX)]],
                    ctx_v.at[b, pl.ds(0, _CTX)], csem[b]).wait()
            pltpu.make_async_copy(table.at[nidx_v.at[pl.ds(0, 2 * _NNEG)]],
                                  neg_v.at[b], nsem[b]).wait()
            @pl.when(p2 >= 1)
            def _():
                pltpu.make_async_copy(odot_hbm.at[pl.ds(0, 2 * _NNEG * 16)],
                                      ndst_v.at[b], wsem[b]).wait()

            cbase = pl.multiple_of(p * (2 * _CTXP), 8)
            for e in range(2):
                i = p * 2 + e
                ibase = pl.multiple_of(cbase + e * _CTXP, 8)
                cacc = jnp.zeros((16,), f32)
                for t in range(3):
                    idt = inp_v[pl.ds(ibase + t * 16, 16)]
                    cacc = cacc + jnp.where(idt != 0, f32(1), f32(0))
                idt = inp_v[pl.ds(ibase + 40, 16)]
                cacc = cacc + jnp.where((lane >= 8) & (lane < 10) & (idt != 0),
                                        f32(1), f32(0))
                cntv = jnp.maximum(jnp.zeros((16,), f32) + jnp.sum(cacc),
                                   jnp.full((16,), f32(1)))
                inv = jnp.full((16,), f32(1)) / cntv

                def row_sum(rI, acc):
                    r2 = pl.multiple_of(rI * 2, 2)
                    a = tuple(
                        acc[cc] + ctx_v[b, e * _CTXP + r2, pl.ds(cc * 16, 16)]
                        for cc in range(8))
                    return tuple(
                        a[cc] + ctx_v[b, e * _CTXP + r2 + 1, pl.ds(cc * 16, 16)]
                        for cc in range(8))

                acc = lax.fori_loop(
                    0, _CTX // 2, row_sum,
                    tuple(jnp.zeros((16,), f32) for _ in range(8)))
                emb = tuple(a * inv for a in acc)

                od = jnp.zeros((16,), f32)
                for cc in range(8):
                    od = od + mwrows_v[i, pl.ds(cc * 16, 16)] * emb[cc]
                odot_v[pl.ds(pl.multiple_of(i * 16, 16), 16)] = od

                for nn in range(_NNEG):
                    nd = jnp.zeros((16,), f32)
                    for cc in range(8):
                        nd = nd + (neg_v[b, e * _NNEG + nn, pl.ds(cc * 16, 16)]
                                   * emb[cc])
                    ndst_v[b, pl.ds((e * _NNEG + nn) * 16, 16)] = nd
            pltpu.async_copy(
                ndst_v.at[b],
                ndot_hbm.at[pl.ds(wid * (_DPW * 16) + p * (2 * _NNEG * 16),
                                  2 * _NNEG * 16)],
                wsem[b])
        return 0

    lax.fori_loop(0, npair // 2, pair_outer, 0)
    for _e in range(2):
        pltpu.make_async_copy(table.at[inp_v.at[pl.ds(0, _CTX)]],
                              ctx_v.at[0, pl.ds(0, _CTX)], csem[0]).wait()
    pltpu.make_async_copy(table.at[nidx_v.at[pl.ds(0, 2 * _NNEG)]],
                          neg_v.at[0], nsem[0]).wait()
    for b in range(2):
        pltpu.make_async_copy(odot_hbm.at[pl.ds(0, 2 * _NNEG * 16)],
                              ndst_v.at[b], wsem[b]).wait()

    pltpu.sync_copy(odot_v, odot_hbm.at[pl.ds(wid * (_BPW * 16), _BPW * 16)])


_sc_main = functools.partial(
    pl.kernel,
    mesh=plsc.VectorSubcoreMesh(core_axis_name="c", subcore_axis_name="s"),
    out_type=[
        jax.ShapeDtypeStruct((_BATCH * 16,), jnp.float32),
        jax.ShapeDtypeStruct((_BATCH * _NNEG * 16,), jnp.float32),
    ],
    scratch_types=[
        pltpu.VMEM((_BPW * _CTXP,), jnp.int32),
        pltpu.VMEM((_DPW,), jnp.float32),
        pltpu.VMEM((_BPW,), jnp.int32),
        pltpu.VMEM((_CPAD,), jnp.float32),
        pltpu.VMEM((_DPW,), jnp.int32),
        pltpu.VMEM((2, 128, 128), jnp.float32),
        pltpu.VMEM((_DPW,), jnp.int32),
        pltpu.VMEM((_BPW, _EMB), jnp.float32),
        pltpu.VMEM((2, 2 * _CTXP, _EMB), jnp.float32),
        pltpu.VMEM((2, 2 * _NNEG, _EMB), jnp.float32),
        pltpu.VMEM((_BPW * 16,), jnp.float32),
        pltpu.VMEM((2, 2 * _NNEG * 16), jnp.float32),
        pltpu.SemaphoreType.DMA,
        pltpu.SemaphoreType.DMA,
        pltpu.SemaphoreType.DMA,
        pltpu.SemaphoreType.DMA,
        pltpu.SemaphoreType.DMA,
        pltpu.SemaphoreType.DMA,
        pltpu.SemaphoreType.DMA,
        pltpu.SemaphoreType.DMA,
        pltpu.SemaphoreType.DMA,
    ],
    compiler_params=pltpu.CompilerParams(needs_layout_passes=False),
)(_sc_body)


def _fin_body(od_ref, nd_ref, out_ref):
    f32 = jnp.float32
    odp = od_ref[:]
    ndp = nd_ref[:]
    ones16 = jnp.ones((16, 1), f32)
    odot = jnp.dot(odp, ones16, preferred_element_type=f32)
    r = lax.broadcasted_iota(jnp.int32, (_NNEG * 16, _NNEG), 0)
    c = lax.broadcasted_iota(jnp.int32, (_NNEG * 16, _NNEG), 1)
    sel = (r // 16 == c).astype(f32)
    ndot = jnp.dot(ndp, sel, preferred_element_type=f32)
    sig_o = f32(1) / (f32(1) + jnp.exp(-odot))
    sig_n = f32(1) / (f32(1) + jnp.exp(ndot))
    oloss = jnp.log(sig_o + f32(1e-5))
    nloss = jnp.mean(jnp.log(sig_n + f32(1e-5)), axis=1, keepdims=True)
    out_ref[:] = -(oloss + nloss)


_fin = pl.pallas_call(
    _fin_body,
    out_shape=jax.ShapeDtypeStruct((_BATCH, 1), jnp.float32),
)


def kernel(input_s, missing_word, lookup_table, weights):
    i32 = jnp.int32
    w2 = jnp.concatenate(
        [weights.astype(jnp.float32),
         jnp.zeros((_VPAD - _VOCAB,), jnp.float32)]).reshape(_ROWS, 128)
    cdf2d, coarse, u, inp56 = _prep(w2, input_s.astype(i32))
    odp, ndp = _sc_main(lookup_table, inp56.reshape(-1), u.reshape(-1),
                        missing_word.astype(i32), coarse.reshape(-1), cdf2d)
    return _fin(odp.reshape(_BATCH, 16),
                ndp.reshape(_BATCH, _NNEG * 16)).reshape(_BATCH)


# speedup vs baseline: 1.0025x; 1.0025x over previous
import functools

import jax
import jax.numpy as jnp
from jax import lax
from jax.experimental import pallas as pl
from jax.experimental.pallas import tpu as pltpu
from jax.experimental.pallas import tpu_sc as plsc

_VOCAB = 100000
_EMB = 128
_BATCH = 4096
_CTX = 50
_NNEG = 20
_CTXP = 56
_ROWS = 784
_VPAD = _ROWS * 128
_CPAD = 1024
_LASTB = 781
_NW = 32
_BPW = _BATCH // _NW
_DPW = _BPW * _NNEG


def _prep_body(w_ref, is_ref, cdf_ref, coarse_ref, u_ref, inp_ref):
    w = w_ref[:]
    f32 = jnp.float32
    r = lax.broadcasted_iota(jnp.int32, (128, 128), 0)
    c = lax.broadcasted_iota(jnp.int32, (128, 128), 1)
    tri = (r <= c).astype(f32)
    rowpref = jnp.dot(w, tri, preferred_element_type=f32)
    rowtot = rowpref[:, 127:128]
    rr = lax.broadcasted_iota(jnp.int32, (_ROWS, _ROWS), 0)
    cc = lax.broadcasted_iota(jnp.int32, (_ROWS, _ROWS), 1)
    lo = (cc <= rr).astype(f32)
    rowcum = jnp.dot(lo, rowtot, preferred_element_type=f32)
    cdf_ref[:] = rowpref + (rowcum - rowtot)
    coarse_ref[:] = jnp.concatenate(
        [rowcum, jnp.full((_CPAD - _ROWS, 1), 2.0, f32)], axis=0)
    pltpu.prng_seed(42)
    bits = pltpu.prng_random_bits((_BATCH, 32))
    bits = lax.bitcast_convert_type(bits, jnp.int32) & jnp.int32(0x7FFFFFFF)
    u_ref[:] = (bits.astype(f32) * f32(2.0 ** -31))[:, :_NNEG]
    isv = is_ref[:]
    inp_ref[:] = jnp.concatenate([isv, isv[:, : _CTXP - _CTX]], axis=1)


_prep = pl.pallas_call(
    _prep_body,
    out_shape=[
        jax.ShapeDtypeStruct((_ROWS, 128), jnp.float32),
        jax.ShapeDtypeStruct((_CPAD, 1), jnp.float32),
        jax.ShapeDtypeStruct((_BATCH, _NNEG), jnp.float32),
        jax.ShapeDtypeStruct((_BATCH, _CTXP), jnp.int32),
    ],
)


def _sc_body(table, inp, uflat, mw, coarse, fine2d, odot_hbm, ndot_hbm,
             inp_v, u_v, mw_v, coarse_v, buck_v, seg_v, nidx_v,
             mwrows_v, ctx_v, neg_v, odot_v, ndst_v,
             mw_sem, fsem0, fsem1, csem0, csem1, nsem0, nsem1, wsem0, wsem1):
    f32 = jnp.float32
    i32 = jnp.int32
    wid = lax.axis_index("s") * 2 + lax.axis_index("c")
    fsem = (fsem0, fsem1)
    csem = (csem0, csem1)
    nsem = (nsem0, nsem1)
    wsem = (wsem0, wsem1)

    pltpu.sync_copy(inp.at[pl.ds(wid * (_BPW * _CTXP), _BPW * _CTXP)], inp_v)
    pltpu.sync_copy(uflat.at[pl.ds(wid * _DPW, _DPW)], u_v)
    pltpu.sync_copy(mw.at[pl.ds(wid * _BPW, _BPW)], mw_v)
    pltpu.sync_copy(coarse, coarse_v)

    pltpu.async_copy(table.at[mw_v], mwrows_v, mw_sem)

    npair = _BPW // 2

    def start_ctx(p, b):
        for e in range(2):
            cb = pl.multiple_of(p * (2 * _CTXP) + e * _CTXP, 8)
            pltpu.async_copy(table.at[inp_v.at[pl.ds(cb, _CTX)]],
                             ctx_v.at[b, pl.ds(e * _CTXP, _CTX)], csem[b])

    def start_neg(p, b):
        nb = pl.multiple_of(p * (2 * _NNEG), 8)
        pltpu.async_copy(table.at[nidx_v.at[pl.ds(nb, 2 * _NNEG)]],
                         neg_v.at[b], nsem[b])

    start_ctx(0, 0)
    start_ctx(1, 1)

    def coarse_chunk(v, _):
        base = pl.multiple_of(v * 16, 16)
        u16 = u_v[pl.ds(base, 16)]
        pos = jnp.zeros((16,), i32)
        s = _CPAD // 2
        while s >= 1:
            probe = pos + (s - 1)
            val = plsc.load_gather(coarse_v, [probe])
            pos = pos + jnp.where(val <= u16, i32(s), i32(0))
            s //= 2
        buck_v[pl.ds(base, 16)] = jnp.minimum(pos, i32(_LASTB))
        return 0

    lax.fori_loop(0, _DPW // 16, coarse_chunk, 0)

    nchunk = _DPW // 128

    def start_fine(cI, b):
        base = pl.multiple_of(cI * 128, 128)
        pltpu.async_copy(fine2d.at[buck_v.at[pl.ds(base, 128)]],
                         seg_v.at[b], fsem[b])

    start_fine(0, 0)

    def fine_outer(f2, _):
        for b in range(2):
            cI = f2 * 2 + b
            start_fine(jnp.minimum(cI + 1, nchunk - 1), 1 - b)
            pltpu.make_async_copy(fine2d.at[buck_v.at[pl.ds(0, 128)]],
                                  seg_v.at[b], fsem[b]).wait()
            base = pl.multiple_of(cI * 128, 128)
            for v in range(8):
                off = pl.multiple_of(base + v * 16, 16)
                u16 = u_v[pl.ds(off, 16)]
                b16 = buck_v[pl.ds(off, 16)]
                rows = v * 16 + lax.iota(i32, 16)
                k = jnp.zeros((16,), i32)
                s = 64
                while s >= 1:
                    probe = k + (s - 1)
                    val = plsc.load_gather(seg_v.at[b], [rows, probe])
                    k = k + jnp.where(val <= u16, i32(s), i32(0))
                    s //= 2
                nidx_v[pl.ds(off, 16)] = jnp.minimum(b16 * 128 + k,
                                                     i32(_VOCAB - 1))
        return 0

    lax.fori_loop(0, nchunk // 2, fine_outer, 0)
    pltpu.make_async_copy(fine2d.at[buck_v.at[pl.ds(0, 128)]],
                          seg_v.at[0], fsem[0]).wait()

    lane = lax.iota(i32, 16)

    start_neg(0, 0)
    pltpu.make_async_copy(table.at[mw_v], mwrows_v, mw_sem).wait()

    def pair_outer(p2, _):
        for b in range(2):
            p = p2 * 2 + b
            start_neg(jnp.minimum(p + 1, npair - 1), 1 - b)
            for _e in range(2):
                pltpu.make_async_copy(
                    table.at[inp_v.at[pl.ds(0, _CTX)]],
                    ctx_v.at[b, pl.ds(0, _CTX)], csem[b]).wait()
            pltpu.make_async_copy(table.at[nidx_v.at[pl.ds(0, 2 * _NNEG)]],
                                  neg_v.at[b], nsem[b]).wait()
            @pl.when(p2 >= 1)
            def _():
                pltpu.make_async_copy(odot_hbm.at[pl.ds(0, 2 * _NNEG * 16)],
                                      ndst_v.at[b], wsem[b]).wait()

            cbase = pl.multiple_of(p * (2 * _CTXP), 8)
            for e in range(2):
                i = p * 2 + e
                ibase = pl.multiple_of(cbase + e * _CTXP, 8)
                cacc = jnp.zeros((16,), f32)
                for t in range(3):
                    idt = inp_v[pl.ds(ibase + t * 16, 16)]
                    cacc = cacc + jnp.where(idt != 0, f32(1), f32(0))
                idt = inp_v[pl.ds(ibase + 40, 16)]
                cacc = cacc + jnp.where((lane >= 8) & (lane < 10) & (idt != 0),
                                        f32(1), f32(0))
                cntv = jnp.maximum(jnp.zeros((16,), f32) + jnp.sum(cacc),
                                   jnp.full((16,), f32(1)))
                inv = jnp.full((16,), f32(1)) / cntv

                def row_sum(rI, acc):
                    r2 = pl.multiple_of(rI * 2, 2)
                    a = tuple(
                        acc[cc] + ctx_v[b, e * _CTXP + r2, pl.ds(cc * 16, 16)]
                        for cc in range(8))
                    return tuple(
                        a[cc] + ctx_v[b, e * _CTXP + r2 + 1, pl.ds(cc * 16, 16)]
                        for cc in range(8))

                acc = lax.fori_loop(
                    0, _CTX // 2, row_sum,
                    tuple(jnp.zeros((16,), f32) for _ in range(8)))
                emb = tuple(a * inv for a in acc)

                od = jnp.zeros((16,), f32)
                for cc in range(8):
                    od = od + mwrows_v[i, pl.ds(cc * 16, 16)] * emb[cc]
                odot_v[pl.ds(pl.multiple_of(i * 16, 16), 16)] = od

                for nn in range(_NNEG):
                    nd = jnp.zeros((16,), f32)
                    for cc in range(8):
                        nd = nd + (neg_v[b, e * _NNEG + nn, pl.ds(cc * 16, 16)]
                                   * emb[cc])
                    ndst_v[b, pl.ds((e * _NNEG + nn) * 16, 16)] = nd
            pltpu.async_copy(
                ndst_v.at[b],
                ndot_hbm.at[pl.ds(wid * (_DPW * 16) + p * (2 * _NNEG * 16),
                                  2 * _NNEG * 16)],
                wsem[b])
            start_ctx(jnp.minimum(p + 2, npair - 1), b)
        return 0

    lax.fori_loop(0, npair // 2, pair_outer, 0)
    for b in range(2):
        for _e in range(2):
            pltpu.make_async_copy(table.at[inp_v.at[pl.ds(0, _CTX)]],
                                  ctx_v.at[b, pl.ds(0, _CTX)], csem[b]).wait()
    pltpu.make_async_copy(table.at[nidx_v.at[pl.ds(0, 2 * _NNEG)]],
                          neg_v.at[0], nsem[0]).wait()
    for b in range(2):
        pltpu.make_async_copy(odot_hbm.at[pl.ds(0, 2 * _NNEG * 16)],
                              ndst_v.at[b], wsem[b]).wait()

    pltpu.sync_copy(odot_v, odot_hbm.at[pl.ds(wid * (_BPW * 16), _BPW * 16)])


_sc_main = functools.partial(
    pl.kernel,
    mesh=plsc.VectorSubcoreMesh(core_axis_name="c", subcore_axis_name="s"),
    out_type=[
        jax.ShapeDtypeStruct((_BATCH * 16,), jnp.float32),
        jax.ShapeDtypeStruct((_BATCH * _NNEG * 16,), jnp.float32),
    ],
    scratch_types=[
        pltpu.VMEM((_BPW * _CTXP,), jnp.int32),
        pltpu.VMEM((_DPW,), jnp.float32),
        pltpu.VMEM((_BPW,), jnp.int32),
        pltpu.VMEM((_CPAD,), jnp.float32),
        pltpu.VMEM((_DPW,), jnp.int32),
        pltpu.VMEM((2, 128, 128), jnp.float32),
        pltpu.VMEM((_DPW,), jnp.int32),
        pltpu.VMEM((_BPW, _EMB), jnp.float32),
        pltpu.VMEM((2, 2 * _CTXP, _EMB), jnp.float32),
        pltpu.VMEM((2, 2 * _NNEG, _EMB), jnp.float32),
        pltpu.VMEM((_BPW * 16,), jnp.float32),
        pltpu.VMEM((2, 2 * _NNEG * 16), jnp.float32),
        pltpu.SemaphoreType.DMA,
        pltpu.SemaphoreType.DMA,
        pltpu.SemaphoreType.DMA,
        pltpu.SemaphoreType.DMA,
        pltpu.SemaphoreType.DMA,
        pltpu.SemaphoreType.DMA,
        pltpu.SemaphoreType.DMA,
        pltpu.SemaphoreType.DMA,
        pltpu.SemaphoreType.DMA,
    ],
    compiler_params=pltpu.CompilerParams(needs_layout_passes=False),
)(_sc_body)


def _fin_body(od_ref, nd_ref, out_ref):
    f32 = jnp.float32
    odp = od_ref[:]
    ndp = nd_ref[:]
    ones16 = jnp.ones((16, 1), f32)
    odot = jnp.dot(odp, ones16, preferred_element_type=f32)
    r = lax.broadcasted_iota(jnp.int32, (_NNEG * 16, _NNEG), 0)
    c = lax.broadcasted_iota(jnp.int32, (_NNEG * 16, _NNEG), 1)
    sel = (r // 16 == c).astype(f32)
    ndot = jnp.dot(ndp, sel, preferred_element_type=f32)
    sig_o = f32(1) / (f32(1) + jnp.exp(-odot))
    sig_n = f32(1) / (f32(1) + jnp.exp(ndot))
    oloss = jnp.log(sig_o + f32(1e-5))
    nloss = jnp.mean(jnp.log(sig_n + f32(1e-5)), axis=1, keepdims=True)
    out_ref[:] = -(oloss + nloss)


_fin = pl.pallas_call(
    _fin_body,
    out_shape=jax.ShapeDtypeStruct((_BATCH, 1), jnp.float32),
)


def kernel(input_s, missing_word, lookup_table, weights):
    i32 = jnp.int32
    w2 = jnp.concatenate(
        [weights.astype(jnp.float32),
         jnp.zeros((_VPAD - _VOCAB,), jnp.float32)]).reshape(_ROWS, 128)
    cdf2d, coarse, u, inp56 = _prep(w2, input_s.astype(i32))
    odp, ndp = _sc_main(lookup_table, inp56.reshape(-1), u.reshape(-1),
                        missing_word.astype(i32), coarse.reshape(-1), cdf2d)
    return _fin(odp.reshape(_BATCH, 16),
                ndp.reshape(_BATCH, _NNEG * 16)).reshape(_BATCH)


# speedup vs baseline: 1.0843x; 1.0817x over previous
import functools

import jax
import jax.numpy as jnp
from jax import lax
from jax.experimental import pallas as pl
from jax.experimental.pallas import tpu as pltpu
from jax.experimental.pallas import tpu_sc as plsc

_VOCAB = 100000
_EMB = 128
_BATCH = 4096
_CTX = 50
_NNEG = 20
_CTXP = 56
_ROWS = 784
_VPAD = _ROWS * 128
_CPAD = 1024
_LASTB = 781
_NW = 32
_BPW = _BATCH // _NW
_DPW = _BPW * _NNEG


def _prep_body(w_ref, is_ref, cdf_ref, coarse_ref, u_ref, inp_ref):
    w = w_ref[:]
    f32 = jnp.float32
    r = lax.broadcasted_iota(jnp.int32, (128, 128), 0)
    c = lax.broadcasted_iota(jnp.int32, (128, 128), 1)
    tri = (r <= c).astype(f32)
    rowpref = jnp.dot(w, tri, preferred_element_type=f32)
    rowtot = rowpref[:, 127:128]
    rr = lax.broadcasted_iota(jnp.int32, (_ROWS, _ROWS), 0)
    cc = lax.broadcasted_iota(jnp.int32, (_ROWS, _ROWS), 1)
    lo = (cc <= rr).astype(f32)
    rowcum = jnp.dot(lo, rowtot, preferred_element_type=f32)
    cdf_ref[:] = rowpref + (rowcum - rowtot)
    coarse_ref[:] = jnp.concatenate(
        [rowcum, jnp.full((_CPAD - _ROWS, 1), 2.0, f32)], axis=0)
    pltpu.prng_seed(42)
    bits = pltpu.prng_random_bits((_BATCH, 32))
    bits = lax.bitcast_convert_type(bits, jnp.int32) & jnp.int32(0x7FFFFFFF)
    u_ref[:] = (bits.astype(f32) * f32(2.0 ** -31))[:, :_NNEG]
    isv = is_ref[:]
    inp_ref[:] = jnp.concatenate([isv, isv[:, : _CTXP - _CTX]], axis=1)


_prep = pl.pallas_call(
    _prep_body,
    out_shape=[
        jax.ShapeDtypeStruct((_ROWS, 128), jnp.float32),
        jax.ShapeDtypeStruct((_CPAD, 1), jnp.float32),
        jax.ShapeDtypeStruct((_BATCH, _NNEG), jnp.float32),
        jax.ShapeDtypeStruct((_BATCH, _CTXP), jnp.int32),
    ],
)


def _sc_body(table, inp, uflat, mw, coarse, fine2d, odot_hbm, ndot_hbm,
             inp_v, u_v, mw_v, coarse_v, buck_v, seg_v, nidx_v,
             mwrows_v, ctx_v, neg_v, odot_v, ndst_v, fshr,
             mw_sem, fsem0, fsem1, csem0, csem1, nsem0, nsem1, wsem0, wsem1):
    f32 = jnp.float32
    i32 = jnp.int32
    wid = lax.axis_index("s") * 2 + lax.axis_index("c")
    fsem = (fsem0, fsem1)
    csem = (csem0, csem1)
    nsem = (nsem0, nsem1)
    wsem = (wsem0, wsem1)

    @pl.when(lax.axis_index("s") == 0)
    def _():
        pltpu.sync_copy(fine2d, fshr)

    pltpu.sync_copy(inp.at[pl.ds(wid * (_BPW * _CTXP), _BPW * _CTXP)], inp_v)
    pltpu.sync_copy(uflat.at[pl.ds(wid * _DPW, _DPW)], u_v)
    pltpu.sync_copy(mw.at[pl.ds(wid * _BPW, _BPW)], mw_v)
    pltpu.sync_copy(coarse, coarse_v)

    pltpu.async_copy(table.at[mw_v], mwrows_v, mw_sem)

    npair = _BPW // 2

    def start_ctx(p, b):
        for e in range(2):
            cb = pl.multiple_of(p * (2 * _CTXP) + e * _CTXP, 8)
            pltpu.async_copy(table.at[inp_v.at[pl.ds(cb, _CTX)]],
                             ctx_v.at[b, pl.ds(e * _CTXP, _CTX)], csem[b])

    def start_neg(p, b):
        nb = pl.multiple_of(p * (2 * _NNEG), 8)
        pltpu.async_copy(table.at[nidx_v.at[pl.ds(nb, 2 * _NNEG)]],
                         neg_v.at[b], nsem[b])

    start_ctx(0, 0)
    start_ctx(1, 1)

    def coarse_chunk(v, _):
        base = pl.multiple_of(v * 16, 16)
        u16 = u_v[pl.ds(base, 16)]
        pos = jnp.zeros((16,), i32)
        s = _CPAD // 2
        while s >= 1:
            probe = pos + (s - 1)
            val = plsc.load_gather(coarse_v, [probe])
            pos = pos + jnp.where(val <= u16, i32(s), i32(0))
            s //= 2
        buck_v[pl.ds(base, 16)] = jnp.minimum(pos, i32(_LASTB))
        return 0

    lax.fori_loop(0, _DPW // 16, coarse_chunk, 0)

    nchunk = _DPW // 128

    def start_fine(cI, b):
        base = pl.multiple_of(cI * 128, 128)
        pltpu.async_copy(fshr.at[buck_v.at[pl.ds(base, 128)]],
                         seg_v.at[b], fsem[b])

    plsc.subcore_barrier()
    start_fine(0, 0)

    def fine_outer(f2, _):
        for b in range(2):
            cI = f2 * 2 + b
            start_fine(jnp.minimum(cI + 1, nchunk - 1), 1 - b)
            pltpu.make_async_copy(fshr.at[buck_v.at[pl.ds(0, 128)]],
                                  seg_v.at[b], fsem[b]).wait()
            base = pl.multiple_of(cI * 128, 128)
            for v in range(8):
                off = pl.multiple_of(base + v * 16, 16)
                u16 = u_v[pl.ds(off, 16)]
                b16 = buck_v[pl.ds(off, 16)]
                rows = v * 16 + lax.iota(i32, 16)
                k = jnp.zeros((16,), i32)
                s = 64
                while s >= 1:
                    probe = k + (s - 1)
                    val = plsc.load_gather(seg_v.at[b], [rows, probe])
                    k = k + jnp.where(val <= u16, i32(s), i32(0))
                    s //= 2
                nidx_v[pl.ds(off, 16)] = jnp.minimum(b16 * 128 + k,
                                                     i32(_VOCAB - 1))
        return 0

    lax.fori_loop(0, nchunk // 2, fine_outer, 0)
    pltpu.make_async_copy(fshr.at[buck_v.at[pl.ds(0, 128)]],
                          seg_v.at[0], fsem[0]).wait()

    lane = lax.iota(i32, 16)

    start_neg(0, 0)
    pltpu.make_async_copy(table.at[mw_v], mwrows_v, mw_sem).wait()

    def pair_outer(p2, _):
        for b in range(2):
            p = p2 * 2 + b
            start_neg(jnp.minimum(p + 1, npair - 1), 1 - b)
            for _e in range(2):
                pltpu.make_async_copy(
                    table.at[inp_v.at[pl.ds(0, _CTX)]],
                    ctx_v.at[b, pl.ds(0, _CTX)], csem[b]).wait()
            pltpu.make_async_copy(table.at[nidx_v.at[pl.ds(0, 2 * _NNEG)]],
                                  neg_v.at[b], nsem[b]).wait()
            @pl.when(p2 >= 1)
            def _():
                pltpu.make_async_copy(odot_hbm.at[pl.ds(0, 2 * _NNEG * 16)],
                                      ndst_v.at[b], wsem[b]).wait()

            cbase = pl.multiple_of(p * (2 * _CTXP), 8)
            for e in range(2):
                i = p * 2 + e
                ibase = pl.multiple_of(cbase + e * _CTXP, 8)
                cacc = jnp.zeros((16,), f32)
                for t in range(3):
                    idt = inp_v[pl.ds(ibase + t * 16, 16)]
                    cacc = cacc + jnp.where(idt != 0, f32(1), f32(0))
                idt = inp_v[pl.ds(ibase + 40, 16)]
                cacc = cacc + jnp.where((lane >= 8) & (lane < 10) & (idt != 0),
                                        f32(1), f32(0))
                cntv = jnp.maximum(jnp.zeros((16,), f32) + jnp.sum(cacc),
                                   jnp.full((16,), f32(1)))
                inv = jnp.full((16,), f32(1)) / cntv

                def row_sum(rI, acc):
                    r2 = pl.multiple_of(rI * 2, 2)
                    a = tuple(
                        acc[cc] + ctx_v[b, e * _CTXP + r2, pl.ds(cc * 16, 16)]
                        for cc in range(8))
                    return tuple(
                        a[cc] + ctx_v[b, e * _CTXP + r2 + 1, pl.ds(cc * 16, 16)]
                        for cc in range(8))

                acc = lax.fori_loop(
                    0, _CTX // 2, row_sum,
                    tuple(jnp.zeros((16,), f32) for _ in range(8)))
                emb = tuple(a * inv for a in acc)

                od = jnp.zeros((16,), f32)
                for cc in range(8):
                    od = od + mwrows_v[i, pl.ds(cc * 16, 16)] * emb[cc]
                odot_v[pl.ds(pl.multiple_of(i * 16, 16), 16)] = od

                for nn in range(_NNEG):
                    nd = jnp.zeros((16,), f32)
                    for cc in range(8):
                        nd = nd + (neg_v[b, e * _NNEG + nn, pl.ds(cc * 16, 16)]
                                   * emb[cc])
                    ndst_v[b, pl.ds((e * _NNEG + nn) * 16, 16)] = nd
            pltpu.async_copy(
                ndst_v.at[b],
                ndot_hbm.at[pl.ds(wid * (_DPW * 16) + p * (2 * _NNEG * 16),
                                  2 * _NNEG * 16)],
                wsem[b])
            start_ctx(jnp.minimum(p + 2, npair - 1), b)
        return 0

    lax.fori_loop(0, npair // 2, pair_outer, 0)
    for b in range(2):
        for _e in range(2):
            pltpu.make_async_copy(table.at[inp_v.at[pl.ds(0, _CTX)]],
                                  ctx_v.at[b, pl.ds(0, _CTX)], csem[b]).wait()
    pltpu.make_async_copy(table.at[nidx_v.at[pl.ds(0, 2 * _NNEG)]],
                          neg_v.at[0], nsem[0]).wait()
    for b in range(2):
        pltpu.make_async_copy(odot_hbm.at[pl.ds(0, 2 * _NNEG * 16)],
                              ndst_v.at[b], wsem[b]).wait()

    pltpu.sync_copy(odot_v, odot_hbm.at[pl.ds(wid * (_BPW * 16), _BPW * 16)])


_sc_main = functools.partial(
    pl.kernel,
    mesh=plsc.VectorSubcoreMesh(core_axis_name="c", subcore_axis_name="s"),
    out_type=[
        jax.ShapeDtypeStruct((_BATCH * 16,), jnp.float32),
        jax.ShapeDtypeStruct((_BATCH * _NNEG * 16,), jnp.float32),
    ],
    scratch_types=[
        pltpu.VMEM((_BPW * _CTXP,), jnp.int32),
        pltpu.VMEM((_DPW,), jnp.float32),
        pltpu.VMEM((_BPW,), jnp.int32),
        pltpu.VMEM((_CPAD,), jnp.float32),
        pltpu.VMEM((_DPW,), jnp.int32),
        pltpu.VMEM((2, 128, 128), jnp.float32),
        pltpu.VMEM((_DPW,), jnp.int32),
        pltpu.VMEM((_BPW, _EMB), jnp.float32),
        pltpu.VMEM((2, 2 * _CTXP, _EMB), jnp.float32),
        pltpu.VMEM((2, 2 * _NNEG, _EMB), jnp.float32),
        pltpu.VMEM((_BPW * 16,), jnp.float32),
        pltpu.VMEM((2, 2 * _NNEG * 16), jnp.float32),
        pltpu.VMEM_SHARED((_ROWS, 128), jnp.float32),
        pltpu.SemaphoreType.DMA,
        pltpu.SemaphoreType.DMA,
        pltpu.SemaphoreType.DMA,
        pltpu.SemaphoreType.DMA,
        pltpu.SemaphoreType.DMA,
        pltpu.SemaphoreType.DMA,
        pltpu.SemaphoreType.DMA,
        pltpu.SemaphoreType.DMA,
        pltpu.SemaphoreType.DMA,
    ],
    compiler_params=pltpu.CompilerParams(needs_layout_passes=False),
)(_sc_body)


def _fin_body(od_ref, nd_ref, out_ref):
    f32 = jnp.float32
    odp = od_ref[:]
    ndp = nd_ref[:]
    ones16 = jnp.ones((16, 1), f32)
    odot = jnp.dot(odp, ones16, preferred_element_type=f32)
    r = lax.broadcasted_iota(jnp.int32, (_NNEG * 16, _NNEG), 0)
    c = lax.broadcasted_iota(jnp.int32, (_NNEG * 16, _NNEG), 1)
    sel = (r // 16 == c).astype(f32)
    ndot = jnp.dot(ndp, sel, preferred_element_type=f32)
    sig_o = f32(1) / (f32(1) + jnp.exp(-odot))
    sig_n = f32(1) / (f32(1) + jnp.exp(ndot))
    oloss = jnp.log(sig_o + f32(1e-5))
    nloss = jnp.mean(jnp.log(sig_n + f32(1e-5)), axis=1, keepdims=True)
    out_ref[:] = -(oloss + nloss)


_fin = pl.pallas_call(
    _fin_body,
    out_shape=jax.ShapeDtypeStruct((_BATCH, 1), jnp.float32),
)


def kernel(input_s, missing_word, lookup_table, weights):
    i32 = jnp.int32
    w2 = jnp.concatenate(
        [weights.astype(jnp.float32),
         jnp.zeros((_VPAD - _VOCAB,), jnp.float32)]).reshape(_ROWS, 128)
    cdf2d, coarse, u, inp56 = _prep(w2, input_s.astype(i32))
    odp, ndp = _sc_main(lookup_table, inp56.reshape(-1), u.reshape(-1),
                        missing_word.astype(i32), coarse.reshape(-1), cdf2d)
    return _fin(odp.reshape(_BATCH, 16),
                ndp.reshape(_BATCH, _NNEG * 16)).reshape(_BATCH)
